# baseline (speedup 1.0000x reference)
import functools

import jax
import jax.numpy as jnp
from jax import lax
from jax.experimental import pallas as pl
from jax.experimental.pallas import tpu as pltpu
from jax.experimental.pallas import tpu_sc as plsc

_N = 10000
_E = 160000
_D = 256
_NC = 2
_NS = 16
_L = 16

_NW = _NC * _NS
_EPT = _E // _NS
_G = 64
_HIST = 10240
_MMB = 2000

_RPT = 313
_ACC_R = 320
_BE = 6400
_NBLK = _E // _BE
_G2 = 32
_SEL2 = 6464
_CAP = 163872
_SEG = 3072

_mesh = plsc.VectorSubcoreMesh(
    core_axis_name="c", subcore_axis_name="s",
    num_cores=_NC, num_subcores=_NS)


def _deg_body(src_hbm, dst_hbm, dout_hbm, din_hbm, idx1, idx2, ones_v, zb,
              hist):
    cid = lax.axis_index("c")
    sid = lax.axis_index("s")
    wid = cid * _NS + sid

    zv = jnp.zeros((_L,), jnp.float32)
    def _zb(k, carry):
        zb[pl.ds(k * _L, _L)] = zv
        return carry
    lax.fori_loop(0, 640 // _L, _zb, 0)
    pltpu.sync_copy(zb, hist.at[pl.ds(sid * 640, 640)])

    ov = jnp.full((_L,), 1.0, jnp.float32)
    for k in range(_G // _L):
        ones_v[pl.ds(k * _L, _L)] = ov

    @pl.when(cid == 0)
    def _():
        pltpu.sync_copy(src_hbm.at[pl.ds(sid * _EPT, _EPT)],
                        idx1.at[pl.ds(0, _EPT)])

    @pl.when(cid == 1)
    def _():
        pltpu.sync_copy(dst_hbm.at[pl.ds(sid * _EPT, _EPT)],
                        idx1.at[pl.ds(0, _EPT)])

    iot = lax.iota(jnp.int32, _L)
    pad_v = iot * 0 + (_N + (wid * 7) % (_HIST - _N))
    for k in range(3):
        idx1[pl.ds(_EPT + k * _L, _L)] = pad_v
    nchunk = (_EPT + 48) // _G
    def _rp(j, carry):
        for c in range(_G // _L):
            idx2[j, pl.ds(c * _L, _L)] = idx1[pl.ds(j * _G + c * _L, _L)]
        return carry
    lax.fori_loop(0, nchunk, _rp, 0)
    plsc.subcore_barrier()

    def _sc(j, carry):
        pltpu.sync_copy(ones_v, hist.at[idx2.at[j]], add=True)
        return carry
    lax.fori_loop(0, nchunk, _sc, 0)
    plsc.subcore_barrier()

    @pl.when(cid == 0)
    def _():
        pltpu.sync_copy(hist.at[pl.ds(sid * 640, 640)],
                        dout_hbm.at[pl.ds(sid * 640, 640)])

    @pl.when(cid == 1)
    def _():
        pltpu.sync_copy(hist.at[pl.ds(sid * 640, 640)],
                        din_hbm.at[pl.ds(sid * 640, 640)])


_deg_call = pl.kernel(
    _deg_body,
    out_type=(jax.ShapeDtypeStruct((_HIST,), jnp.float32),
              jax.ShapeDtypeStruct((_HIST,), jnp.float32)),
    mesh=_mesh,
    scratch_types=[
        pltpu.VMEM((_EPT + 48,), jnp.int32),
        pltpu.VMEM((_EPT // _G + 2, _G), jnp.int32),
        pltpu.VMEM((_G,), jnp.float32),
        pltpu.VMEM((640,), jnp.float32),
        pltpu.VMEM_SHARED((_HIST,), jnp.float32),
    ],
    compiler_params=pltpu.CompilerParams(needs_layout_passes=False),
)


def _prep_body(src_hbm, dst_hbm, lsrc_hbm, ldst_hbm, cnt_hbm,
               src_v, dst_v, selsrc, seldst, cnt_v):
    cid = lax.axis_index("c")
    sid = lax.axis_index("s")
    wid = cid * _NS + sid
    lo = wid * _RPT

    iot = lax.iota(jnp.int32, _L)
    lo_v = iot * 0 + lo
    hi_v = lo_v + _RPT
    one_v = iot * 0 + 1
    psrc_v = iot * 0 + ((wid * 613) % _N)
    pdl_v = iot * 0 + (_RPT + wid % (_ACC_R - _RPT))
    lbase = wid * _CAP

    def _blk(b, off):
        pltpu.sync_copy(src_hbm.at[pl.ds(b * _BE, _BE)], src_v)
        pltpu.sync_copy(dst_hbm.at[pl.ds(b * _BE, _BE)], dst_v)

        def _cb(i, offv):
            vd = dst_v[pl.ds(i * _L, _L)]
            vs = src_v[pl.ds(i * _L, _L)]
            m = (vd >= lo_v) & (vd < hi_v)
            pos = (offv + plsc.cumsum(m.astype(jnp.int32))) - one_v
            plsc.store_scatter(selsrc, [pos], vs, mask=m)
            plsc.store_scatter(seldst, [pos], vd - lo_v, mask=m)
            return offv + plsc.all_reduce_population_count(m)
        offv = lax.fori_loop(0, _BE // _L, _cb, jnp.zeros((_L,), jnp.int32))
        count = jnp.max(offv)

        for k in range(2):
            pp = (count + k * _L) + iot
            plsc.store_scatter(selsrc, [pp], psrc_v)
            plsc.store_scatter(seldst, [pp], pdl_v)

        nch = (count + _G2 - 1) // _G2
        def _ap(j, carry):
            ho = pl.multiple_of(lbase + off + j * _G2, _G2)
            pltpu.sync_copy(selsrc.at[pl.ds(j * _G2, _G2)],
                            lsrc_hbm.at[pl.ds(ho, _G2)])
            pltpu.sync_copy(seldst.at[pl.ds(j * _G2, _G2)],
                            ldst_hbm.at[pl.ds(ho, _G2)])
            return carry
        lax.fori_loop(0, nch, _ap, 0)
        return off + nch * _G2
    off = lax.fori_loop(0, _NBLK, _blk, jnp.int32(0))

    cnt_v[pl.ds(0, _L)] = iot * 0 + off
    pltpu.sync_copy(cnt_v, cnt_hbm.at[pl.ds(wid * _L, _L)])


_prep_call = pl.kernel(
    _prep_body,
    out_type=(jax.ShapeDtypeStruct((_NW * _CAP,), jnp.int32),
              jax.ShapeDtypeStruct((_NW * _CAP,), jnp.int32),
              jax.ShapeDtypeStruct((_NW * _L,), jnp.int32)),
    mesh=_mesh,
    scratch_types=[
        pltpu.VMEM((_BE,), jnp.int32),
        pltpu.VMEM((_BE,), jnp.int32),
        pltpu.VMEM((_SEL2,), jnp.int32),
        pltpu.VMEM((_SEL2,), jnp.int32),
        pltpu.VMEM((_L,), jnp.int32),
    ],
    compiler_params=pltpu.CompilerParams(needs_layout_passes=False),
)


def _agg_body(g_hbm, lsrc_hbm, ldst_hbm, cnt_hbm, out_hbm,
              segsrc, segdst, cnt_v, r0, r1, r2, r3, acc, s0, s1, s2, s3):
    cid = lax.axis_index("c")
    sid = lax.axis_index("s")
    wid = cid * _NS + sid
    rows = (r0, r1, r2, r3)
    sems = (s0, s1, s2, s3)

    zv = jnp.zeros((_L,), jnp.float32)
    def _za(k, carry):
        for u in range(8):
            acc[pl.ds(k * _L * 8 + u * _L, _L)] = zv
        return carry
    lax.fori_loop(0, _ACC_R * _D // _L // 8, _za, 0)

    pltpu.sync_copy(cnt_hbm.at[pl.ds(wid * _L, _L)], cnt_v)
    cnt = cnt_v[pl.ds(0, _L)][0]
    lbase = wid * _CAP

    def _fire(j, rbuf, sem):
        pltpu.async_copy(g_hbm.at[segsrc.at[pl.ds(j * _G2, _G2)]], rbuf, sem)

    def _drain(rbuf, sem):
        pltpu.make_async_copy(g_hbm.at[pl.ds(0, _G2)], rbuf, sem).wait()

    def _accum(j, rbuf):
        for e16 in range(_G2 // _L):
            offs = segdst[pl.ds(j * _G2 + e16 * _L, _L)] * _D
            for e in range(_L):
                off = offs[e]
                for c in range(_D // _L):
                    plsc.addupdate(acc.at[pl.ds(off + c * _L, _L)],
                                   rbuf[e16 * _L + e, pl.ds(c * _L, _L)])

    nseg = (cnt + _SEG - 1) // _SEG
    def _seg(s, carry):
        pltpu.sync_copy(lsrc_hbm.at[pl.ds(lbase + s * _SEG, _SEG)], segsrc)
        pltpu.sync_copy(ldst_hbm.at[pl.ds(lbase + s * _SEG, _SEG)], segdst)
        rem = jnp.minimum(cnt - s * _SEG, _SEG)
        trips = rem // _G2

        for d in range(4):
            @pl.when(d < trips)
            def _():
                _fire(d, rows[d], sems[d])

        def _pq(k, carry2):
            for d in range(4):
                j = 4 * k + d
                @pl.when(j < trips)
                def _():
                    _drain(rows[d], sems[d])
                    _accum(j, rows[d])

                    @pl.when(j + 4 < trips)
                    def _():
                        _fire(j + 4, rows[d], sems[d])
            return carry2
        lax.fori_loop(0, (trips + 3) // 4, _pq, 0)
        return carry
    lax.fori_loop(0, nseg, _seg, 0)

    pltpu.sync_copy(acc.at[pl.ds(0, _RPT * _D)],
                    out_hbm.at[pl.ds(wid * _RPT * _D, _RPT * _D)])


_agg_call = pl.kernel(
    _agg_body,
    out_type=jax.ShapeDtypeStruct((_NW * _RPT * _D,), jnp.float32),
    mesh=_mesh,
    scratch_types=[
        pltpu.VMEM((_SEG,), jnp.int32),
        pltpu.VMEM((_SEG,), jnp.int32),
        pltpu.VMEM((_L,), jnp.int32),
        pltpu.VMEM((_G2, _D), jnp.float32),
        pltpu.VMEM((_G2, _D), jnp.float32),
        pltpu.VMEM((_G2, _D), jnp.float32),
        pltpu.VMEM((_G2, _D), jnp.float32),
        pltpu.VMEM((_ACC_R * _D,), jnp.float32),
        pltpu.SemaphoreType.DMA,
        pltpu.SemaphoreType.DMA,
        pltpu.SemaphoreType.DMA,
        pltpu.SemaphoreType.DMA,
    ],
    compiler_params=pltpu.CompilerParams(needs_layout_passes=False),
)


def _mm_body(a_ref, din_ref, dout_ref, b_ref, w_ref, o_ref):
    ni = lax.rsqrt(jnp.clip(din_ref[...], 1.0, None))
    no = lax.rsqrt(jnp.clip(dout_ref[...], 1.0, None))
    h = a_ref[...] * ni + b_ref[...]
    g = jnp.dot(h, w_ref[...], preferred_element_type=jnp.float32)
    o_ref[...] = g * no


def _fused_mm(a, din2, dout2, b2, w):
    return pl.pallas_call(
        _mm_body,
        grid=(_N // _MMB,),
        in_specs=[
            pl.BlockSpec((_MMB, _D), lambda i: (i, 0)),
            pl.BlockSpec((_MMB, 1), lambda i: (i, 0)),
            pl.BlockSpec((_MMB, 1), lambda i: (i, 0)),
            pl.BlockSpec((1, _D), lambda i: (0, 0)),
            pl.BlockSpec((_D, _D), lambda i: (0, 0)),
        ],
        out_specs=pl.BlockSpec((_MMB, _D), lambda i: (i, 0)),
        out_shape=jax.ShapeDtypeStruct((_N, _D), jnp.float32),
    )(a, din2, dout2, b2, w)


def _fin_body(a_ref, din_ref, b_ref, o_ref):
    ni = lax.rsqrt(jnp.clip(din_ref[...], 1.0, None))
    o_ref[...] = a_ref[...] * ni + b_ref[...]


def _final(a, din2, b2):
    return pl.pallas_call(
        _fin_body,
        grid=(_N // _MMB,),
        in_specs=[
            pl.BlockSpec((_MMB, _D), lambda i: (i, 0)),
            pl.BlockSpec((_MMB, 1), lambda i: (i, 0)),
            pl.BlockSpec((1, _D), lambda i: (0, 0)),
        ],
        out_specs=pl.BlockSpec((_MMB, _D), lambda i: (i, 0)),
        out_shape=jax.ShapeDtypeStruct((_N, _D), jnp.float32),
    )(a, din2, b2)


def _unpad2(a):
    return a.reshape(_NW * _RPT, _D)[:_N]


def kernel(x, edge_index, W1, b1, W2, b2, W3, b3):
    src = edge_index[0].astype(jnp.int32)
    dst = edge_index[1].astype(jnp.int32)

    deg_out, deg_in = _deg_call(src, dst)
    lsrc, ldst, cnts = _prep_call(src, dst)
    din2 = deg_in[:_N].reshape(_N, 1)
    dout2 = deg_out[:_N].reshape(_N, 1)
    ones2 = jnp.ones((_N, 1), jnp.float32)
    zb2 = jnp.zeros((1, _D), jnp.float32)

    din_stack = jnp.stack([ones2, din2, din2])
    b_stack = jnp.stack([zb2, b1.reshape(1, _D), b2.reshape(1, _D)])
    w_stack = jnp.stack([W1, W2, W3])

    def _layer(h, params):
        din_eff, b_prev, w = params
        g = _fused_mm(h, din_eff, dout2, b_prev, w)
        return _unpad2(_agg_call(g, lsrc, ldst, cnts)), None

    h3, _ = lax.scan(_layer, x, (din_stack, b_stack, w_stack))
    return _final(h3, din2, b3.reshape(1, _D))


# speedup vs baseline: 1.0083x; 1.0083x over previous
import jax
import jax.numpy as jnp
from jax import lax
from jax.experimental import pallas as pl
from jax.experimental.pallas import tpu as pltpu
from jax.experimental.pallas import tpu_sc as plsc

_N = 10000
_E = 160000
_D = 256
_NC = 2
_NS = 16
_L = 16

_NW = _NC * _NS
_EPT = _E // _NS
_G = 64
_HIST = 10240
_MMB = 2000

_RPT = 313
_ACC_R = 320
_BE = 6400
_NBLK = _E // _BE
_G2 = 32
_SEL2 = 6464
_CAP = 163872
_SEG = 3072

_mesh = plsc.VectorSubcoreMesh(
    core_axis_name="c", subcore_axis_name="s",
    num_cores=_NC, num_subcores=_NS)


def _deg_body(src_hbm, dst_hbm, dout_hbm, din_hbm, idx1, idx2, ones_v, zb,
              hist):
    cid = lax.axis_index("c")
    sid = lax.axis_index("s")
    wid = cid * _NS + sid

    zv = jnp.zeros((_L,), jnp.float32)
    def _zb(k, carry):
        zb[pl.ds(k * _L, _L)] = zv
        return carry
    lax.fori_loop(0, 640 // _L, _zb, 0)
    pltpu.sync_copy(zb, hist.at[pl.ds(sid * 640, 640)])

    ov = jnp.full((_L,), 1.0, jnp.float32)
    for k in range(_G // _L):
        ones_v[pl.ds(k * _L, _L)] = ov

    @pl.when(cid == 0)
    def _():
        pltpu.sync_copy(src_hbm.at[pl.ds(sid * _EPT, _EPT)],
                        idx1.at[pl.ds(0, _EPT)])

    @pl.when(cid == 1)
    def _():
        pltpu.sync_copy(dst_hbm.at[pl.ds(sid * _EPT, _EPT)],
                        idx1.at[pl.ds(0, _EPT)])

    iot = lax.iota(jnp.int32, _L)
    pad_v = iot * 0 + (_N + (wid * 7) % (_HIST - _N))
    for k in range(3):
        idx1[pl.ds(_EPT + k * _L, _L)] = pad_v
    nchunk = (_EPT + 48) // _G
    def _rp(j, carry):
        for c in range(_G // _L):
            idx2[j, pl.ds(c * _L, _L)] = idx1[pl.ds(j * _G + c * _L, _L)]
        return carry
    lax.fori_loop(0, nchunk, _rp, 0)
    plsc.subcore_barrier()

    def _sc(j, carry):
        pltpu.sync_copy(ones_v, hist.at[idx2.at[j]], add=True)
        return carry
    lax.fori_loop(0, nchunk, _sc, 0)
    plsc.subcore_barrier()

    @pl.when(cid == 0)
    def _():
        pltpu.sync_copy(hist.at[pl.ds(sid * 640, 640)],
                        dout_hbm.at[pl.ds(sid * 640, 640)])

    @pl.when(cid == 1)
    def _():
        pltpu.sync_copy(hist.at[pl.ds(sid * 640, 640)],
                        din_hbm.at[pl.ds(sid * 640, 640)])


_deg_call = pl.kernel(
    _deg_body,
    out_type=(jax.ShapeDtypeStruct((_HIST,), jnp.float32),
              jax.ShapeDtypeStruct((_HIST,), jnp.float32)),
    mesh=_mesh,
    scratch_types=[
        pltpu.VMEM((_EPT + 48,), jnp.int32),
        pltpu.VMEM((_EPT // _G + 2, _G), jnp.int32),
        pltpu.VMEM((_G,), jnp.float32),
        pltpu.VMEM((640,), jnp.float32),
        pltpu.VMEM_SHARED((_HIST,), jnp.float32),
    ],
    compiler_params=pltpu.CompilerParams(needs_layout_passes=False),
)


def _prep_body(src_hbm, dst_hbm, lsrc_hbm, ldst_hbm, cnt_hbm,
               src_v, dst_v, selsrc, seldst, cnt_v):
    cid = lax.axis_index("c")
    sid = lax.axis_index("s")
    wid = cid * _NS + sid
    lo = wid * _RPT

    iot = lax.iota(jnp.int32, _L)
    lo_v = iot * 0 + lo
    hi_v = lo_v + _RPT
    one_v = iot * 0 + 1
    psrc_v = iot * 0 + ((wid * 613) % _N)
    pdl_v = iot * 0 + (_RPT + wid % (_ACC_R - _RPT))
    lbase = wid * _CAP

    def _blk(b, off):
        pltpu.sync_copy(src_hbm.at[pl.ds(b * _BE, _BE)], src_v)
        pltpu.sync_copy(dst_hbm.at[pl.ds(b * _BE, _BE)], dst_v)

        def _cb(i, offv):
            vd = dst_v[pl.ds(i * _L, _L)]
            vs = src_v[pl.ds(i * _L, _L)]
            m = (vd >= lo_v) & (vd < hi_v)
            pos = (offv + plsc.cumsum(m.astype(jnp.int32))) - one_v
            plsc.store_scatter(selsrc, [pos], vs, mask=m)
            plsc.store_scatter(seldst, [pos], vd - lo_v, mask=m)
            return offv + plsc.all_reduce_population_count(m)
        offv = lax.fori_loop(0, _BE // _L, _cb, jnp.zeros((_L,), jnp.int32))
        count = jnp.max(offv)

        for k in range(2):
            pp = (count + k * _L) + iot
            plsc.store_scatter(selsrc, [pp], psrc_v)
            plsc.store_scatter(seldst, [pp], pdl_v)

        nch = (count + _G2 - 1) // _G2
        def _ap(j, carry):
            ho = pl.multiple_of(lbase + off + j * _G2, _G2)
            pltpu.sync_copy(selsrc.at[pl.ds(j * _G2, _G2)],
                            lsrc_hbm.at[pl.ds(ho, _G2)])
            pltpu.sync_copy(seldst.at[pl.ds(j * _G2, _G2)],
                            ldst_hbm.at[pl.ds(ho, _G2)])
            return carry
        lax.fori_loop(0, nch, _ap, 0)
        return off + nch * _G2
    off = lax.fori_loop(0, _NBLK, _blk, jnp.int32(0))

    cnt_v[pl.ds(0, _L)] = iot * 0 + off
    pltpu.sync_copy(cnt_v, cnt_hbm.at[pl.ds(wid * _L, _L)])


_prep_call = pl.kernel(
    _prep_body,
    out_type=(jax.ShapeDtypeStruct((_NW * _CAP,), jnp.int32),
              jax.ShapeDtypeStruct((_NW * _CAP,), jnp.int32),
              jax.ShapeDtypeStruct((_NW * _L,), jnp.int32)),
    mesh=_mesh,
    scratch_types=[
        pltpu.VMEM((_BE,), jnp.int32),
        pltpu.VMEM((_BE,), jnp.int32),
        pltpu.VMEM((_SEL2,), jnp.int32),
        pltpu.VMEM((_SEL2,), jnp.int32),
        pltpu.VMEM((_L,), jnp.int32),
    ],
    compiler_params=pltpu.CompilerParams(needs_layout_passes=False),
)


def _agg_body(g_hbm, lsrc_hbm, ldst_hbm, cnt_hbm, out_hbm,
              segsrc, segdst, cnt_v, r0, r1, r2, r3, acc, s0, s1, s2, s3):
    cid = lax.axis_index("c")
    sid = lax.axis_index("s")
    wid = cid * _NS + sid
    rows = (r0, r1, r2, r3)
    sems = (s0, s1, s2, s3)

    zv = jnp.zeros((_L,), jnp.float32)
    def _za(k, carry):
        for u in range(8):
            acc[pl.ds(k * _L * 8 + u * _L, _L)] = zv
        return carry
    lax.fori_loop(0, _ACC_R * _D // _L // 8, _za, 0)

    pltpu.sync_copy(cnt_hbm.at[pl.ds(wid * _L, _L)], cnt_v)
    cnt = cnt_v[pl.ds(0, _L)][0]
    lbase = wid * _CAP

    def _fire(j, rbuf, sem):
        pltpu.async_copy(g_hbm.at[segsrc.at[pl.ds(j * _G2, _G2)]], rbuf, sem)

    def _drain(rbuf, sem):
        pltpu.make_async_copy(g_hbm.at[pl.ds(0, _G2)], rbuf, sem).wait()

    def _accum(j, rbuf):
        for e16 in range(_G2 // _L):
            offs = segdst[pl.ds(j * _G2 + e16 * _L, _L)] * _D
            for e in range(_L):
                off = offs[e]
                for c in range(_D // _L):
                    plsc.addupdate(acc.at[pl.ds(off + c * _L, _L)],
                                   rbuf[e16 * _L + e, pl.ds(c * _L, _L)])

    nseg = (cnt + _SEG - 1) // _SEG
    def _seg(s, carry):
        pltpu.sync_copy(lsrc_hbm.at[pl.ds(lbase + s * _SEG, _SEG)], segsrc)
        pltpu.sync_copy(ldst_hbm.at[pl.ds(lbase + s * _SEG, _SEG)], segdst)
        rem = jnp.minimum(cnt - s * _SEG, _SEG)
        trips = rem // _G2

        for d in range(4):
            @pl.when(d < trips)
            def _():
                _fire(d, rows[d], sems[d])

        def _pq(k, carry2):
            for d in range(4):
                j = 4 * k + d
                @pl.when(j < trips)
                def _():
                    _drain(rows[d], sems[d])
                    _accum(j, rows[d])

                    @pl.when(j + 4 < trips)
                    def _():
                        _fire(j + 4, rows[d], sems[d])
            return carry2
        lax.fori_loop(0, (trips + 3) // 4, _pq, 0)
        return carry
    lax.fori_loop(0, nseg, _seg, 0)

    pltpu.sync_copy(acc.at[pl.ds(0, _RPT * _D)],
                    out_hbm.at[pl.ds(wid * _RPT * _D, _RPT * _D)])


_agg_call = pl.kernel(
    _agg_body,
    out_type=jax.ShapeDtypeStruct((_NW * _RPT * _D,), jnp.float32),
    mesh=_mesh,
    scratch_types=[
        pltpu.VMEM((_SEG,), jnp.int32),
        pltpu.VMEM((_SEG,), jnp.int32),
        pltpu.VMEM((_L,), jnp.int32),
        pltpu.VMEM((_G2, _D), jnp.float32),
        pltpu.VMEM((_G2, _D), jnp.float32),
        pltpu.VMEM((_G2, _D), jnp.float32),
        pltpu.VMEM((_G2, _D), jnp.float32),
        pltpu.VMEM((_ACC_R * _D,), jnp.float32),
        pltpu.SemaphoreType.DMA,
        pltpu.SemaphoreType.DMA,
        pltpu.SemaphoreType.DMA,
        pltpu.SemaphoreType.DMA,
    ],
    compiler_params=pltpu.CompilerParams(needs_layout_passes=False),
)


def _mm_body(a_ref, din_ref, dout_ref, b_ref, w_ref, o_ref):
    ni = lax.rsqrt(jnp.clip(din_ref[...], 1.0, None))
    no = lax.rsqrt(jnp.clip(dout_ref[...], 1.0, None))
    h = a_ref[...] * ni + b_ref[...]
    g = jnp.dot(h, w_ref[...], preferred_element_type=jnp.float32)
    o_ref[...] = g * no


def _fused_mm(a, din2, dout2, b2, w):
    return pl.pallas_call(
        _mm_body,
        grid=(_N // _MMB,),
        in_specs=[
            pl.BlockSpec((_MMB, _D), lambda i: (i, 0)),
            pl.BlockSpec((_MMB, 1), lambda i: (i, 0)),
            pl.BlockSpec((_MMB, 1), lambda i: (i, 0)),
            pl.BlockSpec((1, _D), lambda i: (0, 0)),
            pl.BlockSpec((_D, _D), lambda i: (0, 0)),
        ],
        out_specs=pl.BlockSpec((_MMB, _D), lambda i: (i, 0)),
        out_shape=jax.ShapeDtypeStruct((_N, _D), jnp.float32),
    )(a, din2, dout2, b2, w)


def _fin_body(a_ref, din_ref, b_ref, o_ref):
    ni = lax.rsqrt(jnp.clip(din_ref[...], 1.0, None))
    o_ref[...] = a_ref[...] * ni + b_ref[...]


def _final(a, din2, b2):
    return pl.pallas_call(
        _fin_body,
        grid=(_N // _MMB,),
        in_specs=[
            pl.BlockSpec((_MMB, _D), lambda i: (i, 0)),
            pl.BlockSpec((_MMB, 1), lambda i: (i, 0)),
            pl.BlockSpec((1, _D), lambda i: (0, 0)),
        ],
        out_specs=pl.BlockSpec((_MMB, _D), lambda i: (i, 0)),
        out_shape=jax.ShapeDtypeStruct((_N, _D), jnp.float32),
    )(a, din2, b2)


def _unpad2(a):
    return a.reshape(_NW * _RPT, _D)[:_N]


def kernel(x, edge_index, W1, b1, W2, b2, W3, b3):
    src = edge_index[0].astype(jnp.int32)
    dst = edge_index[1].astype(jnp.int32)

    deg_out, deg_in = _deg_call(src, dst)
    lsrc, ldst, cnts = _prep_call(src, dst)
    din2 = deg_in[:_N].reshape(_N, 1)
    dout2 = deg_out[:_N].reshape(_N, 1)
    ones2 = jnp.ones((_N, 1), jnp.float32)
    zb2 = jnp.zeros((1, _D), jnp.float32)

    din_stack = jnp.stack([ones2, din2, din2])
    b_stack = jnp.stack([zb2, b1.reshape(1, _D), b2.reshape(1, _D)])
    w_stack = jnp.stack([W1, W2, W3])

    def _layer(h, params):
        din_eff, b_prev, w = params
        g = _fused_mm(h, din_eff, dout2, b_prev, w)
        return _unpad2(_agg_call(g, lsrc, ldst, cnts)), None

    h3, _ = lax.scan(_layer, x, (din_stack, b_stack, w_stack))
    return _final(h3, din2, b3.reshape(1, _D))


# speedup vs baseline: 1.0544x; 1.0457x over previous
import jax
import jax.numpy as jnp
from jax import lax
from jax.experimental import pallas as pl
from jax.experimental.pallas import tpu as pltpu
from jax.experimental.pallas import tpu_sc as plsc

_N = 10000
_E = 160000
_D = 256
_NC = 2
_NS = 16
_L = 16

_NW = _NC * _NS
_EPT = _E // _NS
_G = 64
_HIST = 10240
_MMB = 2000

_RPT = 313
_ACC_R = 320
_BE = 10000
_NBLK = _E // _BE
_G2 = 32
_SEL2 = 10048
_CAP = 163872
_SEG = 6144

_mesh = plsc.VectorSubcoreMesh(
    core_axis_name="c", subcore_axis_name="s",
    num_cores=_NC, num_subcores=_NS)


def _deg_body(src_hbm, dst_hbm, dout_hbm, din_hbm, idx1, idx2, ones_v, zb,
              hist):
    cid = lax.axis_index("c")
    sid = lax.axis_index("s")
    wid = cid * _NS + sid

    zv = jnp.zeros((_L,), jnp.float32)
    def _zb(k, carry):
        zb[pl.ds(k * _L, _L)] = zv
        return carry
    lax.fori_loop(0, 640 // _L, _zb, 0)
    pltpu.sync_copy(zb, hist.at[pl.ds(sid * 640, 640)])

    ov = jnp.full((_L,), 1.0, jnp.float32)
    for k in range(_G // _L):
        ones_v[pl.ds(k * _L, _L)] = ov

    @pl.when(cid == 0)
    def _():
        pltpu.sync_copy(src_hbm.at[pl.ds(sid * _EPT, _EPT)],
                        idx1.at[pl.ds(0, _EPT)])

    @pl.when(cid == 1)
    def _():
        pltpu.sync_copy(dst_hbm.at[pl.ds(sid * _EPT, _EPT)],
                        idx1.at[pl.ds(0, _EPT)])

    iot = lax.iota(jnp.int32, _L)
    pad_v = iot * 0 + (_N + (wid * 7) % (_HIST - _N))
    for k in range(3):
        idx1[pl.ds(_EPT + k * _L, _L)] = pad_v
    nchunk = (_EPT + 48) // _G
    def _rp(j, carry):
        for c in range(_G // _L):
            idx2[j, pl.ds(c * _L, _L)] = idx1[pl.ds(j * _G + c * _L, _L)]
        return carry
    lax.fori_loop(0, nchunk, _rp, 0)
    plsc.subcore_barrier()

    def _sc(j, carry):
        pltpu.sync_copy(ones_v, hist.at[idx2.at[j]], add=True)
        return carry
    lax.fori_loop(0, nchunk, _sc, 0)
    plsc.subcore_barrier()

    @pl.when(cid == 0)
    def _():
        pltpu.sync_copy(hist.at[pl.ds(sid * 640, 640)],
                        dout_hbm.at[pl.ds(sid * 640, 640)])

    @pl.when(cid == 1)
    def _():
        pltpu.sync_copy(hist.at[pl.ds(sid * 640, 640)],
                        din_hbm.at[pl.ds(sid * 640, 640)])


_deg_call = pl.kernel(
    _deg_body,
    out_type=(jax.ShapeDtypeStruct((_HIST,), jnp.float32),
              jax.ShapeDtypeStruct((_HIST,), jnp.float32)),
    mesh=_mesh,
    scratch_types=[
        pltpu.VMEM((_EPT + 48,), jnp.int32),
        pltpu.VMEM((_EPT // _G + 2, _G), jnp.int32),
        pltpu.VMEM((_G,), jnp.float32),
        pltpu.VMEM((640,), jnp.float32),
        pltpu.VMEM_SHARED((_HIST,), jnp.float32),
    ],
    compiler_params=pltpu.CompilerParams(needs_layout_passes=False),
)


def _prep_body(src_hbm, dst_hbm, lsrc_hbm, ldst_hbm, cnt_hbm,
               src_v, dst_v, selsrc, seldst, cnt_v):
    cid = lax.axis_index("c")
    sid = lax.axis_index("s")
    wid = cid * _NS + sid
    lo = wid * _RPT

    iot = lax.iota(jnp.int32, _L)
    lo_v = iot * 0 + lo
    hi_v = lo_v + _RPT
    one_v = iot * 0 + 1
    psrc_v = iot * 0 + ((wid * 613) % _N)
    pdl_v = iot * 0 + (_RPT + wid % (_ACC_R - _RPT))
    lbase = wid * _CAP

    def _blk(b, off):
        pltpu.sync_copy(src_hbm.at[pl.ds(b * _BE, _BE)], src_v)
        pltpu.sync_copy(dst_hbm.at[pl.ds(b * _BE, _BE)], dst_v)

        def _cb(i, offv):
            vd = dst_v[pl.ds(i * _L, _L)]
            vs = src_v[pl.ds(i * _L, _L)]
            m = (vd >= lo_v) & (vd < hi_v)
            pos = (offv + plsc.cumsum(m.astype(jnp.int32))) - one_v
            plsc.store_scatter(selsrc, [pos], vs, mask=m)
            plsc.store_scatter(seldst, [pos], vd - lo_v, mask=m)
            return offv + plsc.all_reduce_population_count(m)
        offv = lax.fori_loop(0, _BE // _L, _cb, jnp.zeros((_L,), jnp.int32))
        count = jnp.max(offv)

        for k in range(2):
            pp = (count + k * _L) + iot
            plsc.store_scatter(selsrc, [pp], psrc_v)
            plsc.store_scatter(seldst, [pp], pdl_v)

        nch = (count + _G2 - 1) // _G2
        def _ap(j, carry):
            ho = pl.multiple_of(lbase + off + j * _G2, _G2)
            pltpu.sync_copy(selsrc.at[pl.ds(j * _G2, _G2)],
                            lsrc_hbm.at[pl.ds(ho, _G2)])
            pltpu.sync_copy(seldst.at[pl.ds(j * _G2, _G2)],
                            ldst_hbm.at[pl.ds(ho, _G2)])
            return carry
        lax.fori_loop(0, nch, _ap, 0)
        return off + nch * _G2
    off = lax.fori_loop(0, _NBLK, _blk, jnp.int32(0))

    cnt_v[pl.ds(0, _L)] = iot * 0 + off
    pltpu.sync_copy(cnt_v, cnt_hbm.at[pl.ds(wid * _L, _L)])


_prep_call = pl.kernel(
    _prep_body,
    out_type=(jax.ShapeDtypeStruct((_NW * _CAP,), jnp.int32),
              jax.ShapeDtypeStruct((_NW * _CAP,), jnp.int32),
              jax.ShapeDtypeStruct((_NW * _L,), jnp.int32)),
    mesh=_mesh,
    scratch_types=[
        pltpu.VMEM((_BE,), jnp.int32),
        pltpu.VMEM((_BE,), jnp.int32),
        pltpu.VMEM((_SEL2,), jnp.int32),
        pltpu.VMEM((_SEL2,), jnp.int32),
        pltpu.VMEM((_L,), jnp.int32),
    ],
    compiler_params=pltpu.CompilerParams(needs_layout_passes=False),
)


def _agg_body(g_hbm, lsrc_hbm, ldst_hbm, cnt_hbm, out_hbm,
              segsrc, segdst, cnt_v, r0, r1, r2, r3, acc, s0, s1, s2, s3):
    cid = lax.axis_index("c")
    sid = lax.axis_index("s")
    wid = cid * _NS + sid
    rows = (r0, r1, r2, r3)
    sems = (s0, s1, s2, s3)

    zv = jnp.zeros((_L,), jnp.float32)
    def _za(k, carry):
        for u in range(8):
            acc[pl.ds(k * _L * 8 + u * _L, _L)] = zv
        return carry
    lax.fori_loop(0, _ACC_R * _D // _L // 8, _za, 0)

    pltpu.sync_copy(cnt_hbm.at[pl.ds(wid * _L, _L)], cnt_v)
    cnt = cnt_v[pl.ds(0, _L)][0]
    lbase = wid * _CAP

    def _fire(j, rbuf, sem):
        pltpu.async_copy(g_hbm.at[segsrc.at[pl.ds(j * _G2, _G2)]], rbuf, sem)

    def _drain(rbuf, sem):
        pltpu.make_async_copy(g_hbm.at[pl.ds(0, _G2)], rbuf, sem).wait()

    def _accum(j, rbuf):
        for e16 in range(_G2 // _L):
            offs = segdst[pl.ds(j * _G2 + e16 * _L, _L)] * _D
            for e in range(_L):
                off = offs[e]
                for c in range(_D // _L):
                    plsc.addupdate(acc.at[pl.ds(off + c * _L, _L)],
                                   rbuf[e16 * _L + e, pl.ds(c * _L, _L)])

    nseg = (cnt + _SEG - 1) // _SEG
    def _seg(s, carry):
        pltpu.sync_copy(lsrc_hbm.at[pl.ds(lbase + s * _SEG, _SEG)], segsrc)
        pltpu.sync_copy(ldst_hbm.at[pl.ds(lbase + s * _SEG, _SEG)], segdst)
        rem = jnp.minimum(cnt - s * _SEG, _SEG)
        trips = rem // _G2

        for d in range(4):
            @pl.when(d < trips)
            def _():
                _fire(d, rows[d], sems[d])

        def _pq(k, carry2):
            for d in range(4):
                j = 4 * k + d
                @pl.when(j < trips)
                def _():
                    _drain(rows[d], sems[d])
                    _accum(j, rows[d])

                    @pl.when(j + 4 < trips)
                    def _():
                        _fire(j + 4, rows[d], sems[d])
            return carry2
        lax.fori_loop(0, (trips + 3) // 4, _pq, 0)
        return carry
    lax.fori_loop(0, nseg, _seg, 0)

    pltpu.sync_copy(acc.at[pl.ds(0, _RPT * _D)],
                    out_hbm.at[pl.ds(wid * _RPT * _D, _RPT * _D)])


_agg_call = pl.kernel(
    _agg_body,
    out_type=jax.ShapeDtypeStruct((_NW * _RPT * _D,), jnp.float32),
    mesh=_mesh,
    scratch_types=[
        pltpu.VMEM((_SEG,), jnp.int32),
        pltpu.VMEM((_SEG,), jnp.int32),
        pltpu.VMEM((_L,), jnp.int32),
        pltpu.VMEM((_G2, _D), jnp.float32),
        pltpu.VMEM((_G2, _D), jnp.float32),
        pltpu.VMEM((_G2, _D), jnp.float32),
        pltpu.VMEM((_G2, _D), jnp.float32),
        pltpu.VMEM((_ACC_R * _D,), jnp.float32),
        pltpu.SemaphoreType.DMA,
        pltpu.SemaphoreType.DMA,
        pltpu.SemaphoreType.DMA,
        pltpu.SemaphoreType.DMA,
    ],
    compiler_params=pltpu.CompilerParams(needs_layout_passes=False),
)


def _mm_body(a_ref, din_ref, dout_ref, b_ref, w_ref, o_ref):
    ni = lax.rsqrt(jnp.clip(din_ref[...], 1.0, None))
    no = lax.rsqrt(jnp.clip(dout_ref[...], 1.0, None))
    h = a_ref[...] * ni + b_ref[...]
    g = jnp.dot(h, w_ref[...], preferred_element_type=jnp.float32)
    o_ref[...] = g * no


def _fused_mm(a, din2, dout2, b2, w):
    return pl.pallas_call(
        _mm_body,
        grid=(_N // _MMB,),
        in_specs=[
            pl.BlockSpec((_MMB, _D), lambda i: (i, 0)),
            pl.BlockSpec((_MMB, 1), lambda i: (i, 0)),
            pl.BlockSpec((_MMB, 1), lambda i: (i, 0)),
            pl.BlockSpec((1, _D), lambda i: (0, 0)),
            pl.BlockSpec((_D, _D), lambda i: (0, 0)),
        ],
        out_specs=pl.BlockSpec((_MMB, _D), lambda i: (i, 0)),
        out_shape=jax.ShapeDtypeStruct((_N, _D), jnp.float32),
    )(a, din2, dout2, b2, w)


def _fin_body(a_ref, din_ref, b_ref, o_ref):
    ni = lax.rsqrt(jnp.clip(din_ref[...], 1.0, None))
    o_ref[...] = a_ref[...] * ni + b_ref[...]


def _final(a, din2, b2):
    return pl.pallas_call(
        _fin_body,
        grid=(_N // _MMB,),
        in_specs=[
            pl.BlockSpec((_MMB, _D), lambda i: (i, 0)),
            pl.BlockSpec((_MMB, 1), lambda i: (i, 0)),
            pl.BlockSpec((1, _D), lambda i: (0, 0)),
        ],
        out_specs=pl.BlockSpec((_MMB, _D), lambda i: (i, 0)),
        out_shape=jax.ShapeDtypeStruct((_N, _D), jnp.float32),
    )(a, din2, b2)


def _unpad2(a):
    return a.reshape(_NW * _RPT, _D)[:_N]


def kernel(x, edge_index, W1, b1, W2, b2, W3, b3):
    src = edge_index[0].astype(jnp.int32)
    dst = edge_index[1].astype(jnp.int32)

    deg_out, deg_in = _deg_call(src, dst)
    lsrc, ldst, cnts = _prep_call(src, dst)
    din2 = deg_in[:_N].reshape(_N, 1)
    dout2 = deg_out[:_N].reshape(_N, 1)
    ones2 = jnp.ones((_N, 1), jnp.float32)
    zb2 = jnp.zeros((1, _D), jnp.float32)

    din_stack = jnp.stack([ones2, din2, din2])
    b_stack = jnp.stack([zb2, b1.reshape(1, _D), b2.reshape(1, _D)])
    w_stack = jnp.stack([W1, W2, W3])

    def _layer(h, params):
        din_eff, b_prev, w = params
        g = _fused_mm(h, din_eff, dout2, b_prev, w)
        return _unpad2(_agg_call(g, lsrc, ldst, cnts)), None

    h3, _ = lax.scan(_layer, x, (din_stack, b_stack, w_stack))
    return _final(h3, din2, b3.reshape(1, _D))
